# Initial kernel scaffold; baseline (speedup 1.0000x reference)
import jax
import jax.numpy as jnp
from jax.experimental import pallas as pl


def kernel(length, item_id, cate_id, emb_item, emb_cate):
    raise NotImplementedError("write your pallas kernel here")


# speedup vs baseline: 4.2000x; 4.2000x over previous
import functools

import jax
import jax.numpy as jnp
from jax import lax
from jax.experimental import pallas as pl
from jax.experimental.pallas import tpu as pltpu
from jax.experimental.pallas import tpu_sc as plsc

_B = 4096
_S = 50
_K = 4
_D = 128
_N = _B * _S

_NC = 2
_NS = 16
_NW = _NC * _NS
_ROWS_W = _N // _NW
_C = 32
_IDX_C = _K * _C
_NCHUNK = _ROWS_W // _C


def _make_encoder():
    mesh = plsc.VectorSubcoreMesh(core_axis_name="c", subcore_axis_name="s")

    @functools.partial(
        pl.kernel,
        mesh=mesh,
        out_type=jax.ShapeDtypeStruct((_N, _D), jnp.float32),
        scratch_types=[
            pltpu.VMEM((_IDX_C,), jnp.int32),
            pltpu.VMEM((_IDX_C,), jnp.int32),
            pltpu.VMEM((_IDX_C, _D), jnp.float32),
            pltpu.VMEM((_IDX_C, _D), jnp.float32),
            pltpu.VMEM((_C, _D), jnp.float32),
            pltpu.SemaphoreType.DMA,
            pltpu.SemaphoreType.DMA,
        ],
    )
    def enc(item_idx_hbm, cate_idx_hbm, emb_item_hbm, emb_cate_hbm, out_hbm,
            ii_v, ci_v, ir_v, cr_v, o_v, sem_i, sem_c):
        wid = lax.axis_index("s") * _NC + lax.axis_index("c")
        row0 = pl.multiple_of(wid * _ROWS_W, _ROWS_W)

        def chunk_body(i, carry):
            base = row0 + i * _C
            ibase = pl.multiple_of(base * _K, _IDX_C)
            pltpu.sync_copy(item_idx_hbm.at[pl.ds(ibase, _IDX_C)], ii_v)
            pltpu.sync_copy(cate_idx_hbm.at[pl.ds(ibase, _IDX_C)], ci_v)
            gi = pltpu.async_copy(emb_item_hbm.at[ii_v], ir_v, sem_i)
            gc = pltpu.async_copy(emb_cate_hbm.at[ci_v], cr_v, sem_c)
            gi.wait()
            gc.wait()

            def row_body(c, carry2):
                r = c * _K
                for d in range(_D // 16):
                    sl = pl.ds(d * 16, 16)
                    acc = (ir_v[r, sl] + ir_v[r + 1, sl]
                           + ir_v[r + 2, sl] + ir_v[r + 3, sl])
                    acc = (acc + cr_v[r, sl] + cr_v[r + 1, sl]
                           + cr_v[r + 2, sl] + cr_v[r + 3, sl])
                    o_v[c, sl] = acc
                return carry2

            lax.fori_loop(0, _C, row_body, 0)
            pltpu.sync_copy(o_v, out_hbm.at[pl.ds(base, _C)])
            return carry

        lax.fori_loop(0, _NCHUNK, chunk_body, 0)

    return enc


_encoder = _make_encoder()


def _mask_body(len_ref, out_ref):
    iota = lax.broadcasted_iota(jnp.int32, (_B, _S), 1)
    out_ref[...] = iota < len_ref[...]


def _seq_mask(length):
    return pl.pallas_call(
        _mask_body,
        out_shape=jax.ShapeDtypeStruct((_B, _S), jnp.bool_),
    )(length.reshape(_B, 1))


def kernel(length, item_id, cate_id, emb_item, emb_cate):
    item_flat = item_id.reshape(_N * _K)
    cate_flat = cate_id.reshape(_N * _K)
    seq = _encoder(item_flat, cate_flat, emb_item, emb_cate)
    seq = seq.reshape(_B, _S, _D)
    return seq, _seq_mask(length)


# speedup vs baseline: 6.5751x; 1.5655x over previous
import functools

import jax
import jax.numpy as jnp
from jax import lax
from jax.experimental import pallas as pl
from jax.experimental.pallas import tpu as pltpu
from jax.experimental.pallas import tpu_sc as plsc

_B = 4096
_S = 50
_K = 4
_D = 128
_N = _B * _S

_NC = 2
_NS = 16
_NW = _NC * _NS
_ROWS_W = _N // _NW
_C = 32
_IDX_C = _K * _C
_NCHUNK = _ROWS_W // _C


def _make_encoder():
    mesh = plsc.VectorSubcoreMesh(core_axis_name="c", subcore_axis_name="s")

    @functools.partial(
        pl.kernel,
        mesh=mesh,
        out_type=jax.ShapeDtypeStruct((_N, _D), jnp.float32),
        scratch_types=[
            pltpu.VMEM((_NCHUNK, _IDX_C), jnp.int32),
            pltpu.VMEM((_NCHUNK, _IDX_C), jnp.int32),
            pltpu.VMEM((_IDX_C, _D), jnp.float32),
            pltpu.VMEM((_IDX_C, _D), jnp.float32),
            pltpu.VMEM((_IDX_C, _D), jnp.float32),
            pltpu.VMEM((_IDX_C, _D), jnp.float32),
            pltpu.VMEM((_C, _D), jnp.float32),
            pltpu.VMEM((_C, _D), jnp.float32),
            pltpu.SemaphoreType.DMA,
            pltpu.SemaphoreType.DMA,
            pltpu.SemaphoreType.DMA,
            pltpu.SemaphoreType.DMA,
        ],
    )
    def enc(item_idx_hbm, cate_idx_hbm, emb_item_hbm, emb_cate_hbm, out_hbm,
            ii_v, ci_v, ir0, ir1, cr0, cr1, o0, o1,
            sg0, sg1, so0, so1):
        wid = lax.axis_index("s") * _NC + lax.axis_index("c")
        row0 = wid * _ROWS_W

        ir = (ir0, ir1)
        cr = (cr0, cr1)
        o = (o0, o1)
        sg = (sg0, sg1)
        so = (so0, so1)

        pltpu.sync_copy(item_idx_hbm.at[pl.ds(wid * _NCHUNK, _NCHUNK)], ii_v)
        pltpu.sync_copy(cate_idx_hbm.at[pl.ds(wid * _NCHUNK, _NCHUNK)], ci_v)

        def issue_gathers(chunk, b):
            pltpu.async_copy(emb_item_hbm.at[ii_v.at[chunk]], ir[b], sg[b])
            pltpu.async_copy(emb_cate_hbm.at[ci_v.at[chunk]], cr[b], sg[b])

        def wait_gathers(b):
            pltpu.make_async_copy(emb_item_hbm.at[ii_v.at[0]], ir[b], sg[b]).wait()
            pltpu.make_async_copy(emb_cate_hbm.at[ci_v.at[0]], cr[b], sg[b]).wait()

        def wait_writeback(b):
            pltpu.make_async_copy(
                o[b], out_hbm.at[pl.ds(row0, _C)], so[b]).wait()

        def compute(b, chunk):
            irb, crb, ob = ir[b], cr[b], o[b]

            def row_body(c, carry):
                r = c * _K
                for d in range(_D // 16):
                    sl = pl.ds(d * 16, 16)
                    acc = (irb[r, sl] + irb[r + 1, sl]
                           + irb[r + 2, sl] + irb[r + 3, sl])
                    acc = (acc + crb[r, sl] + crb[r + 1, sl]
                           + crb[r + 2, sl] + crb[r + 3, sl])
                    ob[c, sl] = acc
                return carry

            lax.fori_loop(0, _C, row_body, 0)
            pltpu.async_copy(ob, out_hbm.at[pl.ds(row0 + chunk * _C, _C)], so[b])

        issue_gathers(0, 0)

        def pair_body(i2, carry):
            chunk = 2 * i2
            issue_gathers(chunk + 1, 1)
            wait_gathers(0)

            @pl.when(i2 >= 1)
            def _():
                wait_writeback(0)
            compute(0, chunk)

            @pl.when(i2 < _NCHUNK // 2 - 1)
            def _():
                issue_gathers(chunk + 2, 0)
            wait_gathers(1)

            @pl.when(i2 >= 1)
            def _():
                wait_writeback(1)
            compute(1, chunk + 1)
            return carry

        lax.fori_loop(0, _NCHUNK // 2, pair_body, 0)
        wait_writeback(0)
        wait_writeback(1)

    return enc


_encoder = _make_encoder()


def _mask_body(len_ref, out_ref):
    iota = lax.broadcasted_iota(jnp.int32, (_B, _S), 1)
    out_ref[...] = iota < len_ref[...]


def _seq_mask(length):
    return pl.pallas_call(
        _mask_body,
        out_shape=jax.ShapeDtypeStruct((_B, _S), jnp.bool_),
    )(length.reshape(_B, 1))


def kernel(length, item_id, cate_id, emb_item, emb_cate):
    item_flat = item_id.reshape(_N * _K // _IDX_C, _IDX_C)
    cate_flat = cate_id.reshape(_N * _K // _IDX_C, _IDX_C)
    seq = _encoder(item_flat, cate_flat, emb_item, emb_cate)
    seq = seq.reshape(_B, _S, _D)
    return seq, _seq_mask(length)


# speedup vs baseline: 6.5790x; 1.0006x over previous
import functools

import jax
import jax.numpy as jnp
from jax import lax
from jax.experimental import pallas as pl
from jax.experimental.pallas import tpu as pltpu
from jax.experimental.pallas import tpu_sc as plsc

_B = 4096
_S = 50
_K = 4
_D = 128
_N = _B * _S

_NC = 2
_NS = 16
_NW = _NC * _NS
_ROWS_W = _N // _NW
_C = 32
_IDX_C = _K * _C
_NCHUNK = _ROWS_W // _C
_NBUF = 3


def _make_encoder():
    mesh = plsc.VectorSubcoreMesh(core_axis_name="c", subcore_axis_name="s")

    row_t = pltpu.VMEM((_IDX_C, _D), jnp.float32)
    idx_t = pltpu.VMEM((_IDX_C,), jnp.int32)
    out_t = pltpu.VMEM((_C, _D), jnp.float32)
    sem_t = pltpu.SemaphoreType.DMA

    @functools.partial(
        pl.kernel,
        mesh=mesh,
        out_type=jax.ShapeDtypeStruct((_N, _D), jnp.float32),
        scratch_types=(
            [idx_t] * _NBUF + [idx_t] * _NBUF
            + [row_t] * _NBUF + [row_t] * _NBUF
            + [out_t] * _NBUF
            + [sem_t] * _NBUF
            + [sem_t] * _NBUF
            + [sem_t] * _NBUF
        ),
    )
    def enc(item_idx_hbm, cate_idx_hbm, emb_item_hbm, emb_cate_hbm, out_hbm,
            ii0, ii1, ii2, ci0, ci1, ci2,
            ir0, ir1, ir2, cr0, cr1, cr2,
            o0, o1, o2,
            si0, si1, si2, sg0, sg1, sg2, so0, so1, so2):
        wid = lax.axis_index("s") * _NC + lax.axis_index("c")
        row0 = wid * _ROWS_W
        chunk0 = wid * _NCHUNK

        ii = (ii0, ii1, ii2)
        ci = (ci0, ci1, ci2)
        ir = (ir0, ir1, ir2)
        cr = (cr0, cr1, cr2)
        o = (o0, o1, o2)
        si = (si0, si1, si2)
        sg = (sg0, sg1, sg2)
        so = (so0, so1, so2)

        def issue_idx(chunk, b):
            pltpu.async_copy(item_idx_hbm.at[chunk0 + chunk], ii[b], si[b])
            pltpu.async_copy(cate_idx_hbm.at[chunk0 + chunk], ci[b], si[b])

        def wait_idx(b):
            pltpu.make_async_copy(item_idx_hbm.at[0], ii[b], si[b]).wait()
            pltpu.make_async_copy(cate_idx_hbm.at[0], ci[b], si[b]).wait()

        def issue_gathers(b):
            pltpu.async_copy(emb_item_hbm.at[ii[b]], ir[b], sg[b])
            pltpu.async_copy(emb_cate_hbm.at[ci[b]], cr[b], sg[b])

        def wait_gathers(b):
            pltpu.make_async_copy(emb_item_hbm.at[ii[b]], ir[b], sg[b]).wait()
            pltpu.make_async_copy(emb_cate_hbm.at[ci[b]], cr[b], sg[b]).wait()

        def wait_writeback(b):
            pltpu.make_async_copy(
                o[b], out_hbm.at[pl.ds(row0, _C)], so[b]).wait()

        def compute(b, chunk):
            irb, crb, ob = ir[b], cr[b], o[b]

            def row_body(c, carry):
                r = c * _K
                for d in range(_D // 16):
                    sl = pl.ds(d * 16, 16)
                    acc = (irb[r, sl] + irb[r + 1, sl]
                           + irb[r + 2, sl] + irb[r + 3, sl])
                    acc = (acc + crb[r, sl] + crb[r + 1, sl]
                           + crb[r + 2, sl] + crb[r + 3, sl])
                    ob[c, sl] = acc
                return carry

            lax.fori_loop(0, _C, row_body, 0)
            pltpu.async_copy(ob, out_hbm.at[pl.ds(row0 + chunk * _C, _C)], so[b])

        def stage(k, bk, wb_guard, next_gather, next_idx):
            if next_gather:
                wait_idx((bk + 1) % _NBUF)
                issue_gathers((bk + 1) % _NBUF)
            if next_idx:
                issue_idx(k + 2, (bk + 2) % _NBUF)
            wait_gathers(bk)
            if wb_guard is None:
                wait_writeback(bk)
            elif wb_guard is not False:
                @pl.when(wb_guard)
                def _():
                    wait_writeback(bk)
            compute(bk, k)

        issue_idx(0, 0)
        issue_idx(1, 1)
        wait_idx(0)
        issue_gathers(0)

        def triple_body(k3, carry):
            guard = k3 >= 1
            for j in range(3):
                stage(3 * k3 + j, j, guard, True, True)
            return carry

        lax.fori_loop(0, (_NCHUNK - 2) // 3, triple_body, 0)

        k = _NCHUNK - 2
        wait_idx(1)
        issue_gathers(1)
        wait_gathers(0)
        wait_writeback(0)
        compute(0, k)
        wait_gathers(1)
        wait_writeback(1)
        compute(1, k + 1)

        wait_writeback(2)
        wait_writeback(0)
        wait_writeback(1)

    return enc


_encoder = _make_encoder()


def _mask_body(len_ref, out_ref):
    iota = lax.broadcasted_iota(jnp.int32, (_B, _S), 1)
    out_ref[...] = iota < len_ref[...]


def _seq_mask(length):
    return pl.pallas_call(
        _mask_body,
        out_shape=jax.ShapeDtypeStruct((_B, _S), jnp.bool_),
    )(length.reshape(_B, 1))


def kernel(length, item_id, cate_id, emb_item, emb_cate):
    item_flat = item_id.reshape(_N * _K // _IDX_C, _IDX_C)
    cate_flat = cate_id.reshape(_N * _K // _IDX_C, _IDX_C)
    seq = _encoder(item_flat, cate_flat, emb_item, emb_cate)
    seq = seq.reshape(_B, _S, _D)
    return seq, _seq_mask(length)


# speedup vs baseline: 7.4631x; 1.1344x over previous
import functools

import jax
import jax.numpy as jnp
from jax import lax
from jax.experimental import pallas as pl
from jax.experimental.pallas import tpu as pltpu
from jax.experimental.pallas import tpu_sc as plsc

_B = 4096
_S = 50
_K = 4
_D = 128

_NC = 2
_NS = 16
_NW = _NC * _NS
_BATCH_W = _B // _NW
_IDX_B = _S * _K
_G0 = 128
_G1 = _IDX_B - _G0


def _make_encoder():
    mesh = plsc.VectorSubcoreMesh(core_axis_name="c", subcore_axis_name="s")

    row_t = pltpu.VMEM((_IDX_B, _D), jnp.float32)
    idx_t = pltpu.VMEM((_IDX_B,), jnp.int32)
    out_t = pltpu.VMEM((_S, _D), jnp.float32)
    sem_t = pltpu.SemaphoreType.DMA

    @functools.partial(
        pl.kernel,
        mesh=mesh,
        out_type=jax.ShapeDtypeStruct((_B, _S, _D), jnp.float32),
        scratch_types=(
            [idx_t, idx_t]
            + [idx_t, idx_t]
            + [row_t, row_t]
            + [row_t, row_t]
            + [out_t, out_t]
            + [sem_t] * 6
        ),
    )
    def enc(item_idx_hbm, cate_idx_hbm, emb_item_hbm, emb_cate_hbm, out_hbm,
            ii0, ii1, ci0, ci1, ir0, ir1, cr0, cr1, o0, o1,
            si0, si1, sg0, sg1, so0, so1):
        wid = lax.axis_index("s") * _NC + lax.axis_index("c")
        b0 = wid * _BATCH_W

        ii = (ii0, ii1)
        ci = (ci0, ci1)
        ir = (ir0, ir1)
        cr = (cr0, cr1)
        o = (o0, o1)
        si = (si0, si1)
        sg = (sg0, sg1)
        so = (so0, so1)

        def issue_idx(k, b):
            off = pl.multiple_of((b0 + k) * _IDX_B, _IDX_B)
            pltpu.async_copy(item_idx_hbm.at[pl.ds(off, _IDX_B)], ii[b], si[b])
            pltpu.async_copy(cate_idx_hbm.at[pl.ds(off, _IDX_B)], ci[b], si[b])

        def wait_idx(b):
            pltpu.make_async_copy(item_idx_hbm.at[pl.ds(0, _IDX_B)], ii[b], si[b]).wait()
            pltpu.make_async_copy(cate_idx_hbm.at[pl.ds(0, _IDX_B)], ci[b], si[b]).wait()

        def issue_gathers(b):
            pltpu.async_copy(emb_item_hbm.at[ii[b].at[pl.ds(0, _G0)]],
                             ir[b].at[pl.ds(0, _G0)], sg[b])
            pltpu.async_copy(emb_item_hbm.at[ii[b].at[pl.ds(_G0, _G1)]],
                             ir[b].at[pl.ds(_G0, _G1)], sg[b])
            pltpu.async_copy(emb_cate_hbm.at[ci[b].at[pl.ds(0, _G0)]],
                             cr[b].at[pl.ds(0, _G0)], sg[b])
            pltpu.async_copy(emb_cate_hbm.at[ci[b].at[pl.ds(_G0, _G1)]],
                             cr[b].at[pl.ds(_G0, _G1)], sg[b])

        def wait_gathers(b):
            pltpu.make_async_copy(emb_item_hbm.at[ii[b].at[pl.ds(0, _G0)]],
                                  ir[b].at[pl.ds(0, _G0)], sg[b]).wait()
            pltpu.make_async_copy(emb_item_hbm.at[ii[b].at[pl.ds(_G0, _G1)]],
                                  ir[b].at[pl.ds(_G0, _G1)], sg[b]).wait()
            pltpu.make_async_copy(emb_cate_hbm.at[ci[b].at[pl.ds(0, _G0)]],
                                  cr[b].at[pl.ds(0, _G0)], sg[b]).wait()
            pltpu.make_async_copy(emb_cate_hbm.at[ci[b].at[pl.ds(_G0, _G1)]],
                                  cr[b].at[pl.ds(_G0, _G1)], sg[b]).wait()

        def wait_writeback(b):
            pltpu.make_async_copy(o[b], out_hbm.at[b0], so[b]).wait()

        def compute(b, k):
            irb, crb, ob = ir[b], cr[b], o[b]

            def row_body(c, carry):
                r = c * _K
                for d in range(_D // 16):
                    sl = pl.ds(d * 16, 16)
                    acc = (irb[r, sl] + irb[r + 1, sl]
                           + irb[r + 2, sl] + irb[r + 3, sl])
                    acc = (acc + crb[r, sl] + crb[r + 1, sl]
                           + crb[r + 2, sl] + crb[r + 3, sl])
                    ob[c, sl] = acc
                return carry

            lax.fori_loop(0, _S, row_body, 0)
            pltpu.async_copy(ob, out_hbm.at[b0 + k], so[b])

        def stage(k, b, wb_guard, next_gather, next_idx):
            if next_gather:
                wait_idx(b ^ 1)
                issue_gathers(b ^ 1)
            wait_gathers(b)
            if next_idx:
                issue_idx(k + 2, b)
            if wb_guard is None:
                wait_writeback(b)
            elif wb_guard is not False:
                @pl.when(wb_guard)
                def _():
                    wait_writeback(b)
            compute(b, k)

        issue_idx(0, 0)
        issue_idx(1, 1)
        wait_idx(0)
        issue_gathers(0)

        def pair_body(k2, carry):
            guard = k2 >= 1
            stage(2 * k2, 0, guard, True, True)
            stage(2 * k2 + 1, 1, guard, True, True)
            return carry

        lax.fori_loop(0, _BATCH_W // 2 - 1, pair_body, 0)

        k = _BATCH_W - 2
        stage(k, 0, None, True, False)
        stage(k + 1, 1, None, False, False)
        wait_writeback(0)
        wait_writeback(1)

    return enc


_encoder = _make_encoder()


def _mask_body(len_ref, out_ref):
    iota = lax.broadcasted_iota(jnp.int32, (_B, _S), 1)
    out_ref[...] = iota < len_ref[...]


def _seq_mask(length):
    return pl.pallas_call(
        _mask_body,
        out_shape=jax.ShapeDtypeStruct((_B, _S), jnp.bool_),
    )(length.reshape(_B, 1))


def kernel(length, item_id, cate_id, emb_item, emb_cate):
    item_flat = item_id.reshape(_B * _S * _K)
    cate_flat = cate_id.reshape(_B * _S * _K)
    seq = _encoder(item_flat, cate_flat, emb_item, emb_cate)
    return seq, _seq_mask(length)
